# baseline (speedup 1.0000x reference)
import jax
import jax.numpy as jnp
from jax.experimental import pallas as pl
from jax.experimental.pallas import tpu as pltpu

_B, _T, _E, _H, _V = 16, 128, 256, 512, 32000
_L = 128

_VMEM_LIM = 55 * 1024 * 1024


def _gather_kernel(idx_ref, tabe_ref, tabd_ref, oute_ref, outd_ref,
                   seme_ref, semd_ref):
    def issue(c, carry):
        base = c * 8
        for i in range(8):
            j = base + i
            ix = idx_ref[j]
            pltpu.make_async_copy(tabe_ref.at[ix], oute_ref.at[j],
                                  seme_ref).start()
            pltpu.make_async_copy(tabd_ref.at[ix], outd_ref.at[j],
                                  semd_ref).start()
        return carry

    jax.lax.fori_loop(0, _B * _T // 8, issue, 0)
    pltpu.make_async_copy(oute_ref, oute_ref, seme_ref).wait()
    pltpu.make_async_copy(outd_ref, outd_ref, semd_ref).wait()


def _embed_gather2(tab_enc3, tab_dec3, idx):
    n = idx.shape[0]
    e = tab_enc3.shape[2]
    return pl.pallas_call(
        _gather_kernel,
        out_shape=(
            jax.ShapeDtypeStruct((n, 1, e), tab_enc3.dtype),
            jax.ShapeDtypeStruct((n, 1, e), tab_dec3.dtype),
        ),
        in_specs=[
            pl.BlockSpec(memory_space=pltpu.SMEM),
            pl.BlockSpec(memory_space=pl.ANY),
            pl.BlockSpec(memory_space=pl.ANY),
        ],
        out_specs=(
            pl.BlockSpec(memory_space=pltpu.VMEM),
            pl.BlockSpec(memory_space=pltpu.VMEM),
        ),
        scratch_shapes=[pltpu.SemaphoreType.DMA, pltpu.SemaphoreType.DMA],
        compiler_params=pltpu.CompilerParams(vmem_limit_bytes=_VMEM_LIM),
        name="embed_gather",
    )(idx, tab_enc3, tab_dec3)


def _scan_kernel(xe_enc_ref, xe_dec_ref,
                 wih_e_ref, bih_e_ref, whh_e_ref, bhh_e_ref,
                 fce_w_ref, fce_b_ref, fcd_w_ref, fcd_b_ref,
                 wih_d_ref, bih_d_ref, whh_d_ref, bhh_d_ref,
                 z_ref, hs_ref, gx_ref):
    H = _H
    B = _B

    def proj_inputs(xe_ref, wih_ref, bih_ref):
        for c in range(_T * _B // 128):
            sl = slice(c * 128, (c + 1) * 128)
            gx_ref[sl, :] = jnp.dot(
                xe_ref[sl, :], wih_ref[...],
                preferred_element_type=jnp.float32) + bih_ref[...]

    def gru_phase(h0, whh_ref, bhh_ref, n_steps, store):
        def step(t, h):
            row = pl.multiple_of(t * B, B)
            g = gx_ref[pl.ds(row, B), :]
            gh = jnp.dot(h.astype(jnp.bfloat16), whh_ref[...],
                         preferred_element_type=jnp.float32) + bhh_ref[...]
            r = jax.nn.sigmoid(g[:, :H] + gh[:, :H])
            u = jax.nn.sigmoid(g[:, H:2 * H] + gh[:, H:2 * H])
            n = jnp.tanh(g[:, 2 * H:] + r * gh[:, 2 * H:])
            h_new = (1.0 - u) * n + u * h
            if store:
                hs_ref[t + 1] = h_new.astype(jnp.bfloat16)
            return h_new

        return jax.lax.fori_loop(0, n_steps, step, h0)

    proj_inputs(xe_enc_ref, wih_e_ref, bih_e_ref)
    h0 = jnp.zeros((B, H), jnp.float32)
    h_last = gru_phase(h0, whh_e_ref, bhh_e_ref, _T, store=False)

    z_val = jnp.dot(h_last.astype(jnp.bfloat16), fce_w_ref[...],
                    preferred_element_type=jnp.float32) + fce_b_ref[...]
    z_ref[...] = z_val
    hid = jnp.tanh(jnp.dot(z_val.astype(jnp.bfloat16), fcd_w_ref[...],
                           preferred_element_type=jnp.float32) + fcd_b_ref[...])

    proj_inputs(xe_dec_ref, wih_d_ref, bih_d_ref)
    hs_ref[0] = jnp.zeros((B, H), jnp.bfloat16)
    gru_phase(hid, whh_d_ref, bhh_d_ref, _T - 1, store=True)


_BM, _BN = 1024, 3200


def _logits_kernel(a_ref, w_ref, b_ref, o_ref):
    w = w_ref[...].astype(jnp.bfloat16)
    acc = jax.lax.dot_general(a_ref[...], w, (((1,), (1,)), ((), ())),
                              preferred_element_type=jnp.float32)
    iota = jax.lax.broadcasted_iota(jnp.int32, (_BM, 1), 0)
    mask = (iota % _T) == 0
    o_ref[...] = jnp.where(mask, 0.0, acc + b_ref[...])


def _logits(a_bf, w_raw, bias):
    m = a_bf.shape[0]
    return pl.pallas_call(
        _logits_kernel,
        out_shape=jax.ShapeDtypeStruct((m, _V), jnp.float32),
        grid=(_V // _BN, m // _BM),
        in_specs=[
            pl.BlockSpec((_BM, _H), lambda i, j: (j, 0)),
            pl.BlockSpec((_BN, _H), lambda i, j: (i, 0)),
            pl.BlockSpec((1, _BN), lambda i, j: (0, i)),
        ],
        out_specs=pl.BlockSpec((_BM, _BN), lambda i, j: (j, i)),
        compiler_params=pltpu.CompilerParams(
            dimension_semantics=("parallel", "arbitrary"),
            vmem_limit_bytes=_VMEM_LIM,
        ),
        name="logits_proj",
    )(a_bf, w_raw, bias)


def kernel(x, enc_emb, enc_W_ih, enc_b_ih, enc_W_hh, enc_b_hh,
           fc_enc_W, fc_enc_b, fc_dec_W, fc_dec_b,
           dec_emb, dec_W_ih, dec_b_ih, dec_W_hh, dec_b_hh,
           dec_fc_W, dec_fc_b):
    B, T = x.shape
    V, E = enc_emb.shape
    H = enc_W_hh.shape[1]
    L = fc_enc_W.shape[0]
    f32, bf16 = jnp.float32, jnp.bfloat16

    x_tm = x.astype(jnp.int32).T.reshape(-1)

    enc_g3, dec_g3 = _embed_gather2(enc_emb.reshape(V, 1, E),
                                    dec_emb.reshape(V, 1, E), x_tm)
    enc_g = enc_g3.reshape(T * B, E)
    dec_g = dec_g3.reshape(T * B, E)

    z, hs = pl.pallas_call(
        _scan_kernel,
        out_shape=(
            jax.ShapeDtypeStruct((B, L), f32),
            jax.ShapeDtypeStruct((T, B, H), bf16),
        ),
        in_specs=[pl.BlockSpec(memory_space=pltpu.VMEM)] * 14,
        out_specs=(
            pl.BlockSpec(memory_space=pltpu.VMEM),
            pl.BlockSpec(memory_space=pltpu.VMEM),
        ),
        scratch_shapes=[pltpu.VMEM((T * B, 3 * H), f32)],
        compiler_params=pltpu.CompilerParams(vmem_limit_bytes=_VMEM_LIM),
        name="gru_scan",
    )(
        enc_g.astype(bf16), dec_g.astype(bf16),
        enc_W_ih.T.astype(bf16), enc_b_ih.reshape(1, -1),
        enc_W_hh.T.astype(bf16), enc_b_hh.reshape(1, -1),
        fc_enc_W.T.astype(bf16), fc_enc_b.reshape(1, -1),
        fc_dec_W.T.astype(bf16), fc_dec_b.reshape(1, -1),
        dec_W_ih.T.astype(bf16), dec_b_ih.reshape(1, -1),
        dec_W_hh.T.astype(bf16), dec_b_hh.reshape(1, -1),
    )

    hs_bt = hs.transpose(1, 0, 2).reshape(B * T, H)
    logits_flat = _logits(hs_bt, dec_fc_W, dec_fc_b.reshape(1, -1))
    outputs = logits_flat.reshape(B, T, V)
    return outputs, z


# speedup vs baseline: 1.6842x; 1.6842x over previous
import jax
import jax.numpy as jnp
from jax.experimental import pallas as pl
from jax.experimental.pallas import tpu as pltpu

_B, _T, _E, _H, _V = 16, 128, 256, 512, 32000
_L = 128

_VMEM_LIM = 55 * 1024 * 1024


def _gather_kernel(idx_ref, tabe_ref, tabd_ref, oute_ref, outd_ref,
                   seme_ref, semd_ref):
    def issue(c, carry):
        base = c * 8
        for i in range(8):
            j = base + i
            ix = idx_ref[j]
            pltpu.make_async_copy(tabe_ref.at[pl.ds(ix, 1), :],
                                  oute_ref.at[pl.ds(j, 1), :],
                                  seme_ref).start()
            pltpu.make_async_copy(tabd_ref.at[pl.ds(ix, 1), :],
                                  outd_ref.at[pl.ds(j, 1), :],
                                  semd_ref).start()
        return carry

    jax.lax.fori_loop(0, _B * _T // 8, issue, 0)
    pltpu.make_async_copy(oute_ref, oute_ref, seme_ref).wait()
    pltpu.make_async_copy(outd_ref, outd_ref, semd_ref).wait()


def _embed_gather2(tab_enc3, tab_dec3, idx):
    n = idx.shape[0]
    e = tab_enc3.shape[1]
    return pl.pallas_call(
        _gather_kernel,
        out_shape=(
            jax.ShapeDtypeStruct((n, e), tab_enc3.dtype),
            jax.ShapeDtypeStruct((n, e), tab_dec3.dtype),
        ),
        in_specs=[
            pl.BlockSpec(memory_space=pltpu.SMEM),
            pl.BlockSpec(memory_space=pl.ANY),
            pl.BlockSpec(memory_space=pl.ANY),
        ],
        out_specs=(
            pl.BlockSpec(memory_space=pltpu.VMEM),
            pl.BlockSpec(memory_space=pltpu.VMEM),
        ),
        scratch_shapes=[pltpu.SemaphoreType.DMA, pltpu.SemaphoreType.DMA],
        compiler_params=pltpu.CompilerParams(vmem_limit_bytes=_VMEM_LIM),
        name="embed_gather",
    )(idx, tab_enc3, tab_dec3)


def _scan_kernel(xe_enc_ref, xe_dec_ref,
                 wih_e_ref, bih_e_ref, whh_e_ref, bhh_e_ref,
                 fce_w_ref, fce_b_ref, fcd_w_ref, fcd_b_ref,
                 wih_d_ref, bih_d_ref, whh_d_ref, bhh_d_ref,
                 z_ref, hs_ref, gx_ref):
    H = _H
    B = _B

    def proj_inputs(xe_ref, wih_ref, bih_ref):
        for c in range(_T * _B // 128):
            sl = slice(c * 128, (c + 1) * 128)
            gx_ref[sl, :] = jnp.dot(
                xe_ref[sl, :], wih_ref[...],
                preferred_element_type=jnp.float32) + bih_ref[...]

    def gru_phase(h0, whh_ref, bhh_ref, n_steps, store):
        def step(t, h):
            row = pl.multiple_of(t * B, B)
            g = gx_ref[pl.ds(row, B), :]
            gh = jnp.dot(h.astype(jnp.bfloat16), whh_ref[...],
                         preferred_element_type=jnp.float32) + bhh_ref[...]
            r = jax.nn.sigmoid(g[:, :H] + gh[:, :H])
            u = jax.nn.sigmoid(g[:, H:2 * H] + gh[:, H:2 * H])
            n = jnp.tanh(g[:, 2 * H:] + r * gh[:, 2 * H:])
            h_new = (1.0 - u) * n + u * h
            if store:
                hs_ref[t + 1] = h_new.astype(jnp.bfloat16)
            return h_new

        def pair(i, h):
            h = step(2 * i, h)
            return step(2 * i + 1, h)

        h = jax.lax.fori_loop(0, n_steps // 2, pair, h0)
        if n_steps % 2:
            h = step(n_steps - 1, h)
        return h

    proj_inputs(xe_enc_ref, wih_e_ref, bih_e_ref)
    h0 = jnp.zeros((B, H), jnp.float32)
    h_last = gru_phase(h0, whh_e_ref, bhh_e_ref, _T, store=False)

    z_val = jnp.dot(h_last.astype(jnp.bfloat16), fce_w_ref[...],
                    preferred_element_type=jnp.float32) + fce_b_ref[...]
    z_ref[...] = z_val
    hid = jnp.tanh(jnp.dot(z_val.astype(jnp.bfloat16), fcd_w_ref[...],
                           preferred_element_type=jnp.float32) + fcd_b_ref[...])

    proj_inputs(xe_dec_ref, wih_d_ref, bih_d_ref)
    hs_ref[0] = jnp.zeros((B, H), jnp.bfloat16)
    gru_phase(hid, whh_d_ref, bhh_d_ref, _T - 1, store=True)


_BM, _BN = 1024, 3200


def _logits_kernel(a_ref, w_ref, b_ref, o_ref):
    w = w_ref[...].astype(jnp.bfloat16)
    acc = jax.lax.dot_general(a_ref[...], w, (((1,), (1,)), ((), ())),
                              preferred_element_type=jnp.float32)
    iota = jax.lax.broadcasted_iota(jnp.int32, (_BM, 1), 0)
    mask = (iota % _T) == 0
    o_ref[...] = jnp.where(mask, 0.0, acc + b_ref[...])


def _logits(a_bf, w_raw, bias):
    m = a_bf.shape[0]
    return pl.pallas_call(
        _logits_kernel,
        out_shape=jax.ShapeDtypeStruct((m, _V), jnp.float32),
        grid=(_V // _BN, m // _BM),
        in_specs=[
            pl.BlockSpec((_BM, _H), lambda i, j: (j, 0)),
            pl.BlockSpec((_BN, _H), lambda i, j: (i, 0)),
            pl.BlockSpec((1, _BN), lambda i, j: (0, i)),
        ],
        out_specs=pl.BlockSpec((_BM, _BN), lambda i, j: (j, i)),
        compiler_params=pltpu.CompilerParams(
            dimension_semantics=("parallel", "arbitrary"),
            vmem_limit_bytes=_VMEM_LIM,
        ),
        name="logits_proj",
    )(a_bf, w_raw, bias)


def kernel(x, enc_emb, enc_W_ih, enc_b_ih, enc_W_hh, enc_b_hh,
           fc_enc_W, fc_enc_b, fc_dec_W, fc_dec_b,
           dec_emb, dec_W_ih, dec_b_ih, dec_W_hh, dec_b_hh,
           dec_fc_W, dec_fc_b):
    B, T = x.shape
    V, E = enc_emb.shape
    H = enc_W_hh.shape[1]
    L = fc_enc_W.shape[0]
    f32, bf16 = jnp.float32, jnp.bfloat16

    x_tm = x.astype(jnp.int32).T.reshape(-1)

    enc_g, dec_g = _embed_gather2(enc_emb, dec_emb, x_tm)

    z, hs = pl.pallas_call(
        _scan_kernel,
        out_shape=(
            jax.ShapeDtypeStruct((B, L), f32),
            jax.ShapeDtypeStruct((T, B, H), bf16),
        ),
        in_specs=[pl.BlockSpec(memory_space=pltpu.VMEM)] * 14,
        out_specs=(
            pl.BlockSpec(memory_space=pltpu.VMEM),
            pl.BlockSpec(memory_space=pltpu.VMEM),
        ),
        scratch_shapes=[pltpu.VMEM((T * B, 3 * H), f32)],
        compiler_params=pltpu.CompilerParams(vmem_limit_bytes=_VMEM_LIM),
        name="gru_scan",
    )(
        enc_g.astype(bf16), dec_g.astype(bf16),
        enc_W_ih.T.astype(bf16), enc_b_ih.reshape(1, -1),
        enc_W_hh.T.astype(bf16), enc_b_hh.reshape(1, -1),
        fc_enc_W.T.astype(bf16), fc_enc_b.reshape(1, -1),
        fc_dec_W.T.astype(bf16), fc_dec_b.reshape(1, -1),
        dec_W_ih.T.astype(bf16), dec_b_ih.reshape(1, -1),
        dec_W_hh.T.astype(bf16), dec_b_hh.reshape(1, -1),
    )

    hs_bt = hs.transpose(1, 0, 2).reshape(B * T, H)
    logits_flat = _logits(hs_bt, dec_fc_W, dec_fc_b.reshape(1, -1))
    outputs = logits_flat.reshape(B, T, V)
    return outputs, z
